# baseline (speedup 1.0000x reference)
import functools

import jax
import jax.numpy as jnp
from jax import lax
from jax.experimental import pallas as pl
from jax.experimental.pallas import tpu as pltpu
from jax.experimental.pallas import tpu_sc as plsc

BATCH = 16384
FIELDS = 26
DIM = 32
TOTAL = BATCH * FIELDS

_info = plsc.get_sparse_core_info()
_NC = _info.num_cores
_NS = _info.num_subcores
_NW = _NC * _NS
_B_PER_W = TOTAL // _NW
_NCHUNK = 8
_C = _B_PER_W // _NCHUNK


NUM_EMB = 1000000
_FULL_BLOCKS = NUM_EMB // 128
_TAIL = NUM_EMB - _FULL_BLOCKS * 128
_K_PER_W = (_FULL_BLOCKS + _NW - 1) // _NW


def _make_transpose_kernel():
    mesh = plsc.VectorSubcoreMesh(core_axis_name="c", subcore_axis_name="s")

    @functools.partial(
        pl.kernel,
        mesh=mesh,
        compiler_params=pltpu.CompilerParams(needs_layout_passes=False),
        out_type=jax.ShapeDtypeStruct((NUM_EMB * DIM,), jnp.float32),
        scratch_types=[
            pltpu.VMEM((DIM, 128), jnp.float32),
            pltpu.VMEM((128 * DIM,), jnp.float32),
            pltpu.SemaphoreType.DMA,
        ],
    )
    def transpose_kernel(wt_hbm, tail_hbm, out_hbm, stage, outbuf, sem):
        wid = lax.axis_index("s") * _NC + lax.axis_index("c")
        iota16 = lax.iota(jnp.int32, 16)

        def do_block(blk):
            for r in range(4):
                pltpu.async_copy(
                    wt_hbm.at[pl.ds(r * 8, 8), pl.ds(blk * 128, 128)],
                    stage.at[pl.ds(r * 8, 8), :],
                    sem,
                ).wait()

            def row_body(i0, carry):
                for u in range(8):
                    i_local = i0 * 8 + u
                    ivec = jnp.broadcast_to(i_local, (16,)).astype(jnp.int32)
                    for h in range(2):
                        vals = plsc.load_gather(
                            stage, [iota16 + (16 * h), ivec])
                        outbuf[pl.ds(i_local * DIM + h * 16, 16)] = vals
                return carry
            lax.fori_loop(0, 16, row_body, 0, unroll=False)

            pltpu.sync_copy(
                outbuf,
                out_hbm.at[pl.ds(blk * 128 * DIM, 128 * DIM)],
            )

        def k_body(k, carry):
            blk = wid + _NW * k

            @pl.when(blk < _FULL_BLOCKS)
            def _do():
                do_block(blk)
            return carry
        lax.fori_loop(0, _K_PER_W, k_body, 0, unroll=False)

        @pl.when(wid == 0)
        def _tail():
            pltpu.sync_copy(tail_hbm, outbuf.at[pl.ds(0, _TAIL * DIM)])
            pltpu.sync_copy(
                outbuf.at[pl.ds(0, _TAIL * DIM)],
                out_hbm.at[pl.ds(_FULL_BLOCKS * 128 * DIM, _TAIL * DIM)],
            )

    return transpose_kernel


def _make_kernel():
    mesh = plsc.VectorSubcoreMesh(core_axis_name="c", subcore_axis_name="s")

    @functools.partial(
        pl.kernel,
        mesh=mesh,
        compiler_params=pltpu.CompilerParams(use_tc_tiling_on_sc=False),
        out_type=jax.ShapeDtypeStruct((TOTAL, DIM), jnp.float32),
        scratch_types=[
            pltpu.VMEM((_NCHUNK, _C), jnp.int32),
            pltpu.VMEM((2, _C, DIM), jnp.float32),
            pltpu.SemaphoreType.DMA,
            pltpu.SemaphoreType.DMA,
            pltpu.SemaphoreType.DMA,
            pltpu.SemaphoreType.DMA,
        ],
    )
    def gather_kernel(idx_hbm, table_hbm, out_hbm, idx_v, rows_v,
                      gsem0, gsem1, osem0, osem1):
        wid = lax.axis_index("s") * _NC + lax.axis_index("c")
        base = wid * _B_PER_W
        gsems = (gsem0, gsem1)
        osems = (osem0, osem1)

        pltpu.sync_copy(idx_hbm.at[wid], idx_v)

        g = [
            pltpu.async_copy(table_hbm.at[idx_v.at[b]], rows_v.at[b], gsems[b])
            for b in range(2)
        ]
        o = [None, None]
        for j in range(_NCHUNK):
            slot = j % 2
            g[slot].wait()
            o[slot] = pltpu.async_copy(
                rows_v.at[slot], out_hbm.at[pl.ds(base + j * _C, _C)],
                osems[slot],
            )
            if j + 2 < _NCHUNK:
                o[slot].wait()
                g[slot] = pltpu.async_copy(
                    table_hbm.at[idx_v.at[j + 2]], rows_v.at[slot],
                    gsems[slot],
                )
        o[0].wait()
        o[1].wait()

    return gather_kernel


_gather = _make_kernel()
_transpose = _make_transpose_kernel()


@jax.jit
def kernel(input, weight):
    idx_flat = input.reshape(_NW, _NCHUNK, _C).astype(jnp.int32)
    tail = lax.slice(weight, (_FULL_BLOCKS * 128, 0), (NUM_EMB, DIM)).reshape(
        _TAIL * DIM)
    w_flat = _transpose(weight.T, tail)
    table = w_flat.reshape(NUM_EMB, DIM)
    out = _gather(idx_flat, table)
    return out.reshape(BATCH, FIELDS, DIM)


# speedup vs baseline: 1.5803x; 1.5803x over previous
import functools

import jax
import jax.numpy as jnp
from jax import lax
from jax.experimental import pallas as pl
from jax.experimental.pallas import tpu as pltpu
from jax.experimental.pallas import tpu_sc as plsc

BATCH = 16384
FIELDS = 26
DIM = 32
TOTAL = BATCH * FIELDS

_info = plsc.get_sparse_core_info()
_NC = _info.num_cores
_NS = _info.num_subcores
_NW = _NC * _NS
_B_PER_W = TOTAL // _NW
_NCHUNK = 8
_C = _B_PER_W // _NCHUNK


NUM_EMB = 1000000
_FULL_BLOCKS = NUM_EMB // 128
_TAIL = NUM_EMB - _FULL_BLOCKS * 128
_K_PER_W = (_FULL_BLOCKS + _NW - 1) // _NW


def _make_transpose_kernel():
    mesh = plsc.VectorSubcoreMesh(core_axis_name="c", subcore_axis_name="s")

    @functools.partial(
        pl.kernel,
        mesh=mesh,
        compiler_params=pltpu.CompilerParams(needs_layout_passes=False),
        out_type=jax.ShapeDtypeStruct((NUM_EMB * DIM,), jnp.float32),
        scratch_types=[
            pltpu.VMEM((2, DIM, 128), jnp.float32),
            pltpu.VMEM((2 * 128 * DIM,), jnp.float32),
            pltpu.SemaphoreType.DMA,
            pltpu.SemaphoreType.DMA,
            pltpu.SemaphoreType.DMA,
            pltpu.SemaphoreType.DMA,
        ],
    )
    def transpose_kernel(wt_hbm, tail_hbm, out_hbm, stage, outbuf,
                         ssem0, ssem1, osem0, osem1):
        wid = lax.axis_index("s") * _NC + lax.axis_index("c")
        iota16 = lax.iota(jnp.int32, 16)
        ssems = (ssem0, ssem1)
        osems = (osem0, osem1)

        def valid(k):
            return wid + _NW * k < _FULL_BLOCKS

        def issue_stage(k, slot):
            blk = wid + _NW * k
            pltpu.async_copy(
                wt_hbm.at[:, pl.ds(blk * 128, 128)], stage.at[slot],
                ssems[slot])

        def wait_stage(slot):
            pltpu.make_async_copy(
                wt_hbm.at[:, pl.ds(0, 128)], stage.at[slot],
                ssems[slot]).wait()

        def issue_out(k, slot):
            blk = wid + _NW * k
            pltpu.async_copy(
                outbuf.at[pl.ds(slot * 128 * DIM, 128 * DIM)],
                out_hbm.at[pl.ds(blk * 128 * DIM, 128 * DIM)], osems[slot])

        def wait_out(slot):
            pltpu.make_async_copy(
                outbuf.at[pl.ds(slot * 128 * DIM, 128 * DIM)],
                out_hbm.at[pl.ds(0, 128 * DIM)],
                osems[slot]).wait()

        def transpose_block(slot):
            st = stage.at[slot]
            ob_base = slot * 128 * DIM

            def row_body(i0, carry):
                for u in range(8):
                    i_local = i0 * 8 + u
                    ivec = jnp.broadcast_to(i_local, (16,)).astype(jnp.int32)
                    for h in range(2):
                        vals = plsc.load_gather(st, [iota16 + (16 * h), ivec])
                        outbuf[pl.ds(ob_base + i_local * DIM + h * 16, 16)] = vals
                return carry
            lax.fori_loop(0, 16, row_body, 0, unroll=4)

        def half(k, j, slot):
            nslot = 1 - slot

            @pl.when(valid(k + 1))
            def _issue():
                issue_stage(k + 1, nslot)

            @pl.when(valid(k))
            def _proc():
                wait_stage(slot)

                @pl.when(j >= 1)
                def _drain():
                    wait_out(slot)
                transpose_block(slot)
                issue_out(k, slot)

        @pl.when(valid(0))
        def _prologue():
            issue_stage(0, 0)

        def k_body(j, carry):
            half(2 * j, j, 0)
            half(2 * j + 1, j, 1)
            return carry
        lax.fori_loop(0, (_K_PER_W + 1) // 2, k_body, 0, unroll=False)

        for k_last in (_K_PER_W - 2, _K_PER_W - 1):
            @pl.when(valid(k_last))
            def _final_drain(k_last=k_last):
                wait_out(k_last % 2)

        @pl.when(wid == 0)
        def _tail():
            pltpu.sync_copy(tail_hbm, outbuf.at[pl.ds(0, _TAIL * DIM)])
            pltpu.sync_copy(
                outbuf.at[pl.ds(0, _TAIL * DIM)],
                out_hbm.at[pl.ds(_FULL_BLOCKS * 128 * DIM, _TAIL * DIM)],
            )

    return transpose_kernel


def _make_kernel():
    mesh = plsc.VectorSubcoreMesh(core_axis_name="c", subcore_axis_name="s")

    @functools.partial(
        pl.kernel,
        mesh=mesh,
        compiler_params=pltpu.CompilerParams(use_tc_tiling_on_sc=False),
        out_type=jax.ShapeDtypeStruct((TOTAL, DIM), jnp.float32),
        scratch_types=[
            pltpu.VMEM((_NCHUNK, _C), jnp.int32),
            pltpu.VMEM((2, _C, DIM), jnp.float32),
            pltpu.SemaphoreType.DMA,
            pltpu.SemaphoreType.DMA,
            pltpu.SemaphoreType.DMA,
            pltpu.SemaphoreType.DMA,
        ],
    )
    def gather_kernel(idx_hbm, table_hbm, out_hbm, idx_v, rows_v,
                      gsem0, gsem1, osem0, osem1):
        wid = lax.axis_index("s") * _NC + lax.axis_index("c")
        base = wid * _B_PER_W
        gsems = (gsem0, gsem1)
        osems = (osem0, osem1)

        pltpu.sync_copy(idx_hbm.at[wid], idx_v)

        g = [
            pltpu.async_copy(table_hbm.at[idx_v.at[b]], rows_v.at[b], gsems[b])
            for b in range(2)
        ]
        o = [None, None]
        for j in range(_NCHUNK):
            slot = j % 2
            g[slot].wait()
            o[slot] = pltpu.async_copy(
                rows_v.at[slot], out_hbm.at[pl.ds(base + j * _C, _C)],
                osems[slot],
            )
            if j + 2 < _NCHUNK:
                o[slot].wait()
                g[slot] = pltpu.async_copy(
                    table_hbm.at[idx_v.at[j + 2]], rows_v.at[slot],
                    gsems[slot],
                )
        o[0].wait()
        o[1].wait()

    return gather_kernel


_gather = _make_kernel()
_transpose = _make_transpose_kernel()


@jax.jit
def kernel(input, weight):
    idx_flat = input.reshape(_NW, _NCHUNK, _C).astype(jnp.int32)
    tail = lax.slice(weight, (_FULL_BLOCKS * 128, 0), (NUM_EMB, DIM)).reshape(
        _TAIL * DIM)
    w_flat = _transpose(weight.T, tail)
    table = w_flat.reshape(NUM_EMB, DIM)
    out = _gather(idx_flat, table)
    return out.reshape(BATCH, FIELDS, DIM)


# speedup vs baseline: 2.1891x; 1.3852x over previous
import functools

import jax
import jax.numpy as jnp
from jax import lax
from jax.experimental import pallas as pl
from jax.experimental.pallas import tpu as pltpu
from jax.experimental.pallas import tpu_sc as plsc

BATCH = 16384
FIELDS = 26
DIM = 32
TOTAL = BATCH * FIELDS

_info = plsc.get_sparse_core_info()
_NC = _info.num_cores
_NS = _info.num_subcores
_NW = _NC * _NS
_B_PER_W = TOTAL // _NW
_NCHUNK = 8
_C = _B_PER_W // _NCHUNK


NUM_EMB = 1000000
_SB = 512
_FULL_BLOCKS = NUM_EMB // _SB
_TAIL = NUM_EMB - _FULL_BLOCKS * _SB
_K_PER_W = (_FULL_BLOCKS + _NW - 1) // _NW


def _make_transpose_kernel():
    mesh = plsc.VectorSubcoreMesh(core_axis_name="c", subcore_axis_name="s")

    @functools.partial(
        pl.kernel,
        mesh=mesh,
        compiler_params=pltpu.CompilerParams(needs_layout_passes=False),
        out_type=jax.ShapeDtypeStruct((NUM_EMB * DIM,), jnp.float32),
        scratch_types=[
            pltpu.VMEM((2, DIM, _SB + 1), jnp.float32),
            pltpu.VMEM((2 * _SB * DIM,), jnp.float32),
            pltpu.SemaphoreType.DMA,
            pltpu.SemaphoreType.DMA,
            pltpu.SemaphoreType.DMA,
            pltpu.SemaphoreType.DMA,
        ],
    )
    def transpose_kernel(wt_hbm, tail_hbm, out_hbm, stage, outbuf,
                         ssem0, ssem1, osem0, osem1):
        wid = lax.axis_index("s") * _NC + lax.axis_index("c")
        iota16 = lax.iota(jnp.int32, 16)
        ssems = (ssem0, ssem1)
        osems = (osem0, osem1)

        def valid(k):
            return wid + _NW * k < _FULL_BLOCKS

        def issue_stage(k, slot):
            blk = wid + _NW * k
            pltpu.async_copy(
                wt_hbm.at[:, pl.ds(blk * _SB, _SB)],
                stage.at[slot, :, pl.ds(0, _SB)],
                ssems[slot])

        def wait_stage(slot):
            pltpu.make_async_copy(
                wt_hbm.at[:, pl.ds(0, _SB)],
                stage.at[slot, :, pl.ds(0, _SB)],
                ssems[slot]).wait()

        def issue_out(k, slot):
            blk = wid + _NW * k
            pltpu.async_copy(
                outbuf.at[pl.ds(slot * _SB * DIM, _SB * DIM)],
                out_hbm.at[pl.ds(blk * _SB * DIM, _SB * DIM)], osems[slot])

        def wait_out(slot):
            pltpu.make_async_copy(
                outbuf.at[pl.ds(slot * _SB * DIM, _SB * DIM)],
                out_hbm.at[pl.ds(0, _SB * DIM)],
                osems[slot]).wait()

        def transpose_block(slot):
            st = stage.at[slot]
            ob_base = slot * _SB * DIM

            def row_body(i0, carry):
                bvec = jnp.broadcast_to(i0 * 32, (16,)).astype(jnp.int32)
                s_base = ob_base + i0 * (32 * DIM)
                for u4 in range(8):
                    vals8 = []
                    for u in range(u4 * 4, u4 * 4 + 4):
                        ivec = bvec + u
                        for h in range(2):
                            vals8.append((
                                u, h,
                                plsc.load_gather(st, [iota16 + (16 * h), ivec]),
                            ))
                    for u, h, vals in vals8:
                        outbuf[pl.ds(s_base + u * DIM + h * 16, 16)] = vals
                return carry
            lax.fori_loop(0, _SB // 32, row_body, 0, unroll=False)

        def step(k, j, slot):
            @pl.when(valid(k + 1))
            def _issue():
                issue_stage(k + 1, 1 - slot)

            @pl.when((j >= 1) & valid(k - 2))
            def _drain():
                wait_out(slot)

            @pl.when(valid(k))
            def _proc():
                wait_stage(slot)
                transpose_block(slot)
                issue_out(k, slot)

        @pl.when(valid(0))
        def _prologue():
            issue_stage(0, 0)

        n_rounds = (_K_PER_W + 1) // 2
        def k_body(j, carry):
            for t in range(2):
                step(2 * j + t, j, t)
            return carry
        lax.fori_loop(0, n_rounds, k_body, 0, unroll=False)

        for k_last in range(2 * n_rounds - 2, 2 * n_rounds):
            @pl.when(valid(k_last))
            def _final_drain(k_last=k_last):
                wait_out(k_last % 2)

        @pl.when(wid == 0)
        def _tail():
            pltpu.sync_copy(tail_hbm, outbuf.at[pl.ds(0, _TAIL * DIM)])
            pltpu.sync_copy(
                outbuf.at[pl.ds(0, _TAIL * DIM)],
                out_hbm.at[pl.ds(_FULL_BLOCKS * _SB * DIM, _TAIL * DIM)],
            )

    return transpose_kernel


def _make_kernel():
    mesh = plsc.VectorSubcoreMesh(core_axis_name="c", subcore_axis_name="s")

    @functools.partial(
        pl.kernel,
        mesh=mesh,
        compiler_params=pltpu.CompilerParams(use_tc_tiling_on_sc=False),
        out_type=jax.ShapeDtypeStruct((TOTAL, DIM), jnp.float32),
        scratch_types=[
            pltpu.VMEM((_NCHUNK, _C), jnp.int32),
            pltpu.VMEM((2, _C, DIM), jnp.float32),
            pltpu.SemaphoreType.DMA,
            pltpu.SemaphoreType.DMA,
            pltpu.SemaphoreType.DMA,
            pltpu.SemaphoreType.DMA,
        ],
    )
    def gather_kernel(idx_hbm, table_hbm, out_hbm, idx_v, rows_v,
                      gsem0, gsem1, osem0, osem1):
        wid = lax.axis_index("s") * _NC + lax.axis_index("c")
        base = wid * _B_PER_W
        gsems = (gsem0, gsem1)
        osems = (osem0, osem1)

        pltpu.sync_copy(idx_hbm.at[wid], idx_v)

        g = [
            pltpu.async_copy(table_hbm.at[idx_v.at[b]], rows_v.at[b], gsems[b])
            for b in range(2)
        ]
        o = [None, None]
        for j in range(_NCHUNK):
            slot = j % 2
            g[slot].wait()
            o[slot] = pltpu.async_copy(
                rows_v.at[slot], out_hbm.at[pl.ds(base + j * _C, _C)],
                osems[slot],
            )
            if j + 2 < _NCHUNK:
                o[slot].wait()
                g[slot] = pltpu.async_copy(
                    table_hbm.at[idx_v.at[j + 2]], rows_v.at[slot],
                    gsems[slot],
                )
        o[0].wait()
        o[1].wait()

    return gather_kernel


_gather = _make_kernel()
_transpose = _make_transpose_kernel()


@jax.jit
def kernel(input, weight):
    idx_flat = input.reshape(_NW, _NCHUNK, _C).astype(jnp.int32)
    tail = lax.slice(weight, (_FULL_BLOCKS * _SB, 0), (NUM_EMB, DIM)).reshape(
        _TAIL * DIM)
    w_flat = _transpose(weight.T, tail)
    table = w_flat.reshape(NUM_EMB, DIM)
    out = _gather(idx_flat, table)
    return out.reshape(BATCH, FIELDS, DIM)
